# baseline (speedup 1.0000x reference)
import jax
import jax.numpy as jnp
from jax.experimental import pallas as pl
from jax.experimental.pallas import tpu as pltpu

N = 10000
D_IN = 128
D_OUT = 128
BM = 200
HALF = N // 2
CH = 2000
INV_N = 1.0 / N


def _gcn_body(adja_ref, adjb_ref, x_ref, w_ref, g_ref, bta_ref, out_ref,
              y_ref, sup_ref, s1_ref, s2_ref, stage_ref, sem):
    i = pl.program_id(0)

    @pl.when(i == 0)
    def _():
        sup_ref[...] = jnp.dot(
            x_ref[...], w_ref[...], preferred_element_type=jnp.float32
        )
        s1_ref[...] = jnp.zeros_like(s1_ref)
        s2_ref[...] = jnp.zeros_like(s2_ref)

    blka = jnp.dot(adja_ref[...], sup_ref[...], preferred_element_type=jnp.float32)
    y_ref[pl.ds(i * BM, BM), :] = blka
    blkb = jnp.dot(adjb_ref[...], sup_ref[...], preferred_element_type=jnp.float32)
    y_ref[pl.ds(HALF + i * BM, BM), :] = blkb
    s1_ref[...] += (jnp.sum(blka, axis=0, keepdims=True)
                    + jnp.sum(blkb, axis=0, keepdims=True))
    s2_ref[...] += (jnp.sum(blka * blka, axis=0, keepdims=True)
                    + jnp.sum(blkb * blkb, axis=0, keepdims=True))

    @pl.when(i == pl.num_programs(0) - 1)
    def _():
        mean = s1_ref[...] * INV_N
        var = s2_ref[...] * INV_N - mean * mean
        scale = jax.lax.rsqrt(var + 1e-5) * g_ref[...]
        shift = bta_ref[...] - mean * scale

        n_chunks = N // CH
        for c in range(n_chunks):
            buf = c % 2
            z = y_ref[pl.ds(c * CH, CH), :] * scale + shift
            stage_ref[buf] = jnp.where(z >= 0, z, 0.01 * z)
            pltpu.make_async_copy(
                stage_ref.at[buf],
                out_ref.at[pl.ds(c * CH, CH), :],
                sem.at[buf],
            ).start()
            if c >= 1:
                pltpu.make_async_copy(
                    stage_ref.at[(c - 1) % 2],
                    out_ref.at[pl.ds((c - 1) * CH, CH), :],
                    sem.at[(c - 1) % 2],
                ).wait()
        pltpu.make_async_copy(
            stage_ref.at[(n_chunks - 1) % 2],
            out_ref.at[pl.ds((n_chunks - 1) * CH, CH), :],
            sem.at[(n_chunks - 1) % 2],
        ).wait()


def kernel(input, adj, W, b, gamma, beta):
    del b
    g2 = gamma.reshape(1, D_OUT)
    bt2 = beta.reshape(1, D_OUT)
    half_blocks = HALF // BM
    grid = (half_blocks,)
    return pl.pallas_call(
        _gcn_body,
        grid=grid,
        in_specs=[
            pl.BlockSpec((BM, N), lambda i: (i, 0)),
            pl.BlockSpec((BM, N), lambda i: (i + HALF // BM, 0)),
            pl.BlockSpec((N, D_IN), lambda i: (0, 0)),
            pl.BlockSpec((D_IN, D_OUT), lambda i: (0, 0)),
            pl.BlockSpec((1, D_OUT), lambda i: (0, 0)),
            pl.BlockSpec((1, D_OUT), lambda i: (0, 0)),
        ],
        out_specs=pl.BlockSpec(memory_space=pl.ANY),
        out_shape=jax.ShapeDtypeStruct((N, D_OUT), jnp.float32),
        scratch_shapes=[
            pltpu.VMEM((N, D_OUT), jnp.float32),
            pltpu.VMEM((N, D_IN), jnp.float32),
            pltpu.VMEM((1, D_OUT), jnp.float32),
            pltpu.VMEM((1, D_OUT), jnp.float32),
            pltpu.VMEM((2, CH, D_OUT), jnp.float32),
            pltpu.SemaphoreType.DMA((2,)),
        ],
    )(adj, adj, input, W, g2, bt2)


# speedup vs baseline: 1.0153x; 1.0153x over previous
import jax
import jax.numpy as jnp
from jax.experimental import pallas as pl
from jax.experimental.pallas import tpu as pltpu

N = 10000
D_IN = 128
D_OUT = 128
BM = 400
CH = 2000
INV_N = 1.0 / N


def _gcn_body(adj_ref, x_ref, w_ref, g_ref, bta_ref, out_ref,
              y_ref, sup_ref, s1_ref, s2_ref, stage_ref, sem):
    i = pl.program_id(0)

    @pl.when(i == 0)
    def _():
        sup_ref[...] = jnp.dot(
            x_ref[...], w_ref[...], preferred_element_type=jnp.float32
        )
        s1_ref[...] = jnp.zeros_like(s1_ref)
        s2_ref[...] = jnp.zeros_like(s2_ref)

    blk = jnp.dot(adj_ref[...], sup_ref[...], preferred_element_type=jnp.float32)
    y_ref[pl.ds(i * BM, BM), :] = blk
    s1_ref[...] += jnp.sum(blk, axis=0, keepdims=True)
    s2_ref[...] += jnp.sum(blk * blk, axis=0, keepdims=True)

    @pl.when(i == pl.num_programs(0) - 1)
    def _():
        mean = s1_ref[...] * INV_N
        var = s2_ref[...] * INV_N - mean * mean
        scale = jax.lax.rsqrt(var + 1e-5) * g_ref[...]
        shift = bta_ref[...] - mean * scale

        n_chunks = N // CH
        for c in range(n_chunks):
            buf = c % 2
            z = y_ref[pl.ds(c * CH, CH), :] * scale + shift
            stage_ref[buf] = jnp.where(z >= 0, z, 0.01 * z)
            pltpu.make_async_copy(
                stage_ref.at[buf],
                out_ref.at[pl.ds(c * CH, CH), :],
                sem.at[buf],
            ).start()
            if c >= 1:
                pltpu.make_async_copy(
                    stage_ref.at[(c - 1) % 2],
                    out_ref.at[pl.ds((c - 1) * CH, CH), :],
                    sem.at[(c - 1) % 2],
                ).wait()
        pltpu.make_async_copy(
            stage_ref.at[(n_chunks - 1) % 2],
            out_ref.at[pl.ds((n_chunks - 1) * CH, CH), :],
            sem.at[(n_chunks - 1) % 2],
        ).wait()


def kernel(input, adj, W, b, gamma, beta):
    del b
    g2 = gamma.reshape(1, D_OUT)
    bt2 = beta.reshape(1, D_OUT)
    grid = (N // BM,)
    return pl.pallas_call(
        _gcn_body,
        grid=grid,
        in_specs=[
            pl.BlockSpec((BM, N), lambda i: (i, 0)),
            pl.BlockSpec((N, D_IN), lambda i: (0, 0)),
            pl.BlockSpec((D_IN, D_OUT), lambda i: (0, 0)),
            pl.BlockSpec((1, D_OUT), lambda i: (0, 0)),
            pl.BlockSpec((1, D_OUT), lambda i: (0, 0)),
        ],
        out_specs=pl.BlockSpec(memory_space=pl.ANY),
        out_shape=jax.ShapeDtypeStruct((N, D_OUT), jnp.float32),
        scratch_shapes=[
            pltpu.VMEM((N, D_OUT), jnp.float32),
            pltpu.VMEM((N, D_IN), jnp.float32),
            pltpu.VMEM((1, D_OUT), jnp.float32),
            pltpu.VMEM((1, D_OUT), jnp.float32),
            pltpu.VMEM((2, CH, D_OUT), jnp.float32),
            pltpu.SemaphoreType.DMA((2,)),
        ],
    )(adj, input, W, g2, bt2)
